# baseline (speedup 1.0000x reference)
import functools

import jax
import jax.numpy as jnp
from jax import lax
from jax.experimental import pallas as pl
from jax.experimental.pallas import tpu as pltpu
from jax.experimental.pallas import tpu_sc as plsc

N = 8192
D_IN = 512
D_HID = 2048
D_MODEL = 2048
TILE = 512
NT = N // TILE + 1
M = NT * TILE

_R = 64
_C = 128


def _rank_body(t_ref, rank_ref, c0p_ref):
    t = t_ref[...]
    is0 = (t == 0).astype(jnp.float32)
    k = lax.broadcasted_iota(jnp.int32, (_C, _C), 0)
    j = lax.broadcasted_iota(jnp.int32, (_C, _C), 1)
    upper = (k <= j).astype(jnp.float32)
    cum_lane = lax.dot_general(is0, upper, (((1,), (0,)), ((), ())),
                               preferred_element_type=jnp.float32)
    row_tot = cum_lane[:, _C - 1:_C]
    i2 = lax.broadcasted_iota(jnp.int32, (_R, _R), 0)
    j2 = lax.broadcasted_iota(jnp.int32, (_R, _R), 1)
    lower = (j2 < i2).astype(jnp.float32)
    row_off = lax.dot_general(lower, row_tot, (((1,), (0,)), ((), ())),
                              preferred_element_type=jnp.float32)
    cum0 = (cum_lane + row_off).astype(jnp.int32)
    c0 = jnp.sum(is0).astype(jnp.int32)
    c0p = ((c0 + TILE - 1) // TILE) * TILE
    gi = (lax.broadcasted_iota(jnp.int32, (_R, _C), 0) * _C
          + lax.broadcasted_iota(jnp.int32, (_R, _C), 1))
    rank_ref[...] = jnp.where(t == 0, cum0 - 1, c0p + gi - cum0)
    c0p_ref[0, 0] = c0p


def _compute_rank(types2d):
    return pl.pallas_call(
        _rank_body,
        out_shape=(jax.ShapeDtypeStruct((_R, _C), jnp.int32),
                   jax.ShapeDtypeStruct((1, 1), jnp.int32)),
        out_specs=(pl.BlockSpec(memory_space=pltpu.VMEM),
                   pl.BlockSpec(memory_space=pltpu.SMEM)),
    )(types2d)


_NC = 2
_NS = 16
_NW = _NC * _NS
_RPW = N // _NW
_SCAT_CHUNK = 64
_GATH_CHUNK = 16

@functools.lru_cache(maxsize=None)
def _make_sc_scatter():
    mesh = plsc.VectorSubcoreMesh(core_axis_name="c", subcore_axis_name="s")

    @functools.partial(
        pl.kernel, mesh=mesh,
        out_type=jax.ShapeDtypeStruct((M, D_IN), jnp.float32),
        scratch_types=[
            pltpu.VMEM((_SCAT_CHUNK,), jnp.int32),
            pltpu.VMEM((_SCAT_CHUNK,), jnp.int32),
            pltpu.VMEM((_SCAT_CHUNK, D_IN), jnp.float32),
            pltpu.VMEM((_SCAT_CHUNK, D_IN), jnp.float32),
            pltpu.SemaphoreType.DMA,
            pltpu.SemaphoreType.DMA,
            pltpu.SemaphoreType.DMA,
            pltpu.SemaphoreType.DMA,
            pltpu.SemaphoreType.DMA,
            pltpu.SemaphoreType.DMA,
        ],
    )
    def _sc_scatter(x_hbm, rank_hbm, xs_hbm, idx0, idx1, rows0, rows1,
                    si0, si1, sx0, sx1, ss0, ss1):
        wid = lax.axis_index("s") * _NC + lax.axis_index("c")
        base = wid * _RPW
        nch = _RPW // _SCAT_CHUNK
        idx = (idx0, idx1)
        rows = (rows0, rows1)
        semi = (si0, si1)
        semx = (sx0, sx1)
        sems = (ss0, ss1)

        def load(c, b):
            off = base + c * _SCAT_CHUNK
            return (
                pltpu.async_copy(rank_hbm.at[pl.ds(off, _SCAT_CHUNK)],
                                 idx[b], semi[b]),
                pltpu.async_copy(x_hbm.at[pl.ds(off, _SCAT_CHUNK)],
                                 rows[b], semx[b]),
            )

        ld = [load(0, 0), load(1, 1)]
        sd = [None, None]
        for c in range(nch):
            b = c & 1
            nb = b ^ 1
            ld[b][0].wait()
            ld[b][1].wait()
            sd[b] = pltpu.async_copy(rows[b], xs_hbm.at[idx[b]], sems[b])
            if c >= 1 and c + 1 < nch:
                sd[nb].wait()
                sd[nb] = None
                ld[nb] = load(c + 1, nb)
        for d in sd:
            if d is not None:
                d.wait()

    return _sc_scatter


@functools.lru_cache(maxsize=None)
def _make_sc_gather():
    mesh = plsc.VectorSubcoreMesh(core_axis_name="c", subcore_axis_name="s")
    nch = _RPW // _GATH_CHUNK

    @functools.partial(
        pl.kernel, mesh=mesh,
        out_type=jax.ShapeDtypeStruct((N, D_MODEL), jnp.float32),
        scratch_types=[
            pltpu.VMEM((_GATH_CHUNK,), jnp.int32),
            pltpu.VMEM((_GATH_CHUNK,), jnp.int32),
            pltpu.VMEM((_GATH_CHUNK, D_MODEL), jnp.float32),
            pltpu.VMEM((_GATH_CHUNK, D_MODEL), jnp.float32),
            pltpu.SemaphoreType.DMA,
            pltpu.SemaphoreType.DMA,
            pltpu.SemaphoreType.DMA,
            pltpu.SemaphoreType.DMA,
            pltpu.SemaphoreType.DMA,
            pltpu.SemaphoreType.DMA,
        ],
    )
    def _sc_gather(enc_hbm, rank_hbm, out_hbm, idx0, idx1, rows0, rows1,
                   si0, si1, sg0, sg1, so0, so1):
        wid = lax.axis_index("s") * _NC + lax.axis_index("c")
        base = wid * _RPW
        idx = (idx0, idx1)
        rows = (rows0, rows1)
        semi = (si0, si1)
        semg = (sg0, sg1)
        semo = (so0, so1)

        def load_idx(c, b):
            return pltpu.async_copy(
                rank_hbm.at[pl.ds(base + c * _GATH_CHUNK, _GATH_CHUNK)],
                idx[b], semi[b])

        il = [load_idx(0, 0), load_idx(1, 1)]
        gd = [None, None]
        wd = [None, None]
        il[0].wait()
        gd[0] = pltpu.async_copy(enc_hbm.at[idx[0]], rows[0], semg[0])
        for c in range(nch):
            b = c & 1
            nb = b ^ 1
            if c + 1 < nch:
                il[nb].wait()
                if wd[nb] is not None:
                    wd[nb].wait()
                gd[nb] = pltpu.async_copy(enc_hbm.at[idx[nb]], rows[nb],
                                          semg[nb])
            gd[b].wait()
            if c + 2 < nch:
                il[b] = load_idx(c + 2, b)
            wd[b] = pltpu.async_copy(
                rows[b],
                out_hbm.at[pl.ds(base + c * _GATH_CHUNK, _GATH_CHUNK)],
                semo[b])
        wd[0].wait()
        wd[1].wait()

    return _sc_gather


def _mlp_body(eid_ref, xs_ref, w0_ref, b0_ref, w1_ref, b1_ref, out_ref):
    del eid_ref
    xt = xs_ref[...].astype(jnp.bfloat16)
    h = jnp.maximum(
        lax.dot_general(xt, w0_ref[0], (((1,), (0,)), ((), ())),
                        preferred_element_type=jnp.float32) + b0_ref[0], 0.0)
    out_ref[...] = lax.dot_general(
        h.astype(jnp.bfloat16), w1_ref[0], (((1,), (0,)), ((), ())),
        preferred_element_type=jnp.float32) + b1_ref[0]


def _run_mlp(expert_ids, xs, w0s, b0s, w1s, b1s):
    grid_spec = pltpu.PrefetchScalarGridSpec(
        num_scalar_prefetch=1,
        grid=(NT,),
        in_specs=[
            pl.BlockSpec((TILE, D_IN), lambda i, eid: (i, 0)),
            pl.BlockSpec((1, D_IN, D_HID), lambda i, eid: (eid[i], 0, 0)),
            pl.BlockSpec((1, 1, D_HID), lambda i, eid: (eid[i], 0, 0)),
            pl.BlockSpec((1, D_HID, D_MODEL), lambda i, eid: (eid[i], 0, 0)),
            pl.BlockSpec((1, 1, D_MODEL), lambda i, eid: (eid[i], 0, 0)),
        ],
        out_specs=pl.BlockSpec((TILE, D_MODEL), lambda i, eid: (i, 0)),
    )
    return pl.pallas_call(
        _mlp_body,
        grid_spec=grid_spec,
        out_shape=jax.ShapeDtypeStruct((M, D_MODEL), jnp.float32),
        compiler_params=pltpu.CompilerParams(
            dimension_semantics=("parallel",)),
    )(expert_ids, xs, w0s, b0s, w1s, b1s)


def kernel(x, types, W0_0, b0_0, W1_0, b1_0, W0_1, b0_1, W1_1, b1_1):
    types2d = types.astype(jnp.int32).reshape(_R, _C)
    rank2d, c0p = _compute_rank(types2d)
    rank = rank2d.reshape(N)
    expert_ids = (jnp.arange(NT, dtype=jnp.int32) * TILE >= c0p[0, 0]
                  ).astype(jnp.int32)
    xs = _make_sc_scatter()(x, rank)
    w0s = jnp.stack([W0_0, W0_1]).astype(jnp.bfloat16)
    b0s = jnp.stack([b0_0, b0_1]).reshape(2, 1, D_HID)
    w1s = jnp.stack([W1_0, W1_1]).astype(jnp.bfloat16)
    b1s = jnp.stack([b1_0, b1_1]).reshape(2, 1, D_MODEL)
    enc = _run_mlp(expert_ids, xs, w0s, b0s, w1s, b1s)
    return _make_sc_gather()(enc, rank)


# speedup vs baseline: 1.0028x; 1.0028x over previous
import functools

import jax
import jax.numpy as jnp
from jax import lax
from jax.experimental import pallas as pl
from jax.experimental.pallas import tpu as pltpu
from jax.experimental.pallas import tpu_sc as plsc

N = 8192
D_IN = 512
D_HID = 2048
D_MODEL = 2048
TILE = 512
NT = N // TILE + 1
M = NT * TILE

_R = 64
_C = 128


def _rank_body(t_ref, rank_ref, c0p_ref):
    t = t_ref[...]
    is0 = (t == 0).astype(jnp.float32)
    k = lax.broadcasted_iota(jnp.int32, (_C, _C), 0)
    j = lax.broadcasted_iota(jnp.int32, (_C, _C), 1)
    upper = (k <= j).astype(jnp.float32)
    cum_lane = lax.dot_general(is0, upper, (((1,), (0,)), ((), ())),
                               preferred_element_type=jnp.float32)
    row_tot = cum_lane[:, _C - 1:_C]
    i2 = lax.broadcasted_iota(jnp.int32, (_R, _R), 0)
    j2 = lax.broadcasted_iota(jnp.int32, (_R, _R), 1)
    lower = (j2 < i2).astype(jnp.float32)
    row_off = lax.dot_general(lower, row_tot, (((1,), (0,)), ((), ())),
                              preferred_element_type=jnp.float32)
    cum0 = (cum_lane + row_off).astype(jnp.int32)
    c0 = jnp.sum(is0).astype(jnp.int32)
    c0p = ((c0 + TILE - 1) // TILE) * TILE
    gi = (lax.broadcasted_iota(jnp.int32, (_R, _C), 0) * _C
          + lax.broadcasted_iota(jnp.int32, (_R, _C), 1))
    rank_ref[...] = jnp.where(t == 0, cum0 - 1, c0p + gi - cum0)
    c0p_ref[0, 0] = c0p


def _compute_rank(types2d):
    return pl.pallas_call(
        _rank_body,
        out_shape=(jax.ShapeDtypeStruct((_R, _C), jnp.int32),
                   jax.ShapeDtypeStruct((1, 1), jnp.int32)),
        out_specs=(pl.BlockSpec(memory_space=pltpu.VMEM),
                   pl.BlockSpec(memory_space=pltpu.SMEM)),
    )(types2d)


_NC = 2
_NS = 16
_NW = _NC * _NS
_RPW = N // _NW
_SCAT_CHUNK = 64
_GATH_CHUNK = 16

@functools.lru_cache(maxsize=None)
def _make_sc_scatter():
    mesh = plsc.VectorSubcoreMesh(core_axis_name="c", subcore_axis_name="s")

    @functools.partial(
        pl.kernel, mesh=mesh,
        out_type=jax.ShapeDtypeStruct((M, D_IN), jnp.float32),
        scratch_types=[
            pltpu.VMEM((_SCAT_CHUNK,), jnp.int32),
            pltpu.VMEM((_SCAT_CHUNK,), jnp.int32),
            pltpu.VMEM((_SCAT_CHUNK, D_IN), jnp.float32),
            pltpu.VMEM((_SCAT_CHUNK, D_IN), jnp.float32),
            pltpu.SemaphoreType.DMA,
            pltpu.SemaphoreType.DMA,
            pltpu.SemaphoreType.DMA,
            pltpu.SemaphoreType.DMA,
            pltpu.SemaphoreType.DMA,
            pltpu.SemaphoreType.DMA,
        ],
    )
    def _sc_scatter(x_hbm, rank_hbm, xs_hbm, idx0, idx1, rows0, rows1,
                    si0, si1, sx0, sx1, ss0, ss1):
        wid = lax.axis_index("s") * _NC + lax.axis_index("c")
        base = wid * _RPW
        nch = _RPW // _SCAT_CHUNK
        idx = (idx0, idx1)
        rows = (rows0, rows1)
        semi = (si0, si1)
        semx = (sx0, sx1)
        sems = (ss0, ss1)

        def load(c, b):
            off = base + c * _SCAT_CHUNK
            return (
                pltpu.async_copy(rank_hbm.at[pl.ds(off, _SCAT_CHUNK)],
                                 idx[b], semi[b]),
                pltpu.async_copy(x_hbm.at[pl.ds(off, _SCAT_CHUNK)],
                                 rows[b], semx[b]),
            )

        ld = [load(0, 0), load(1, 1)]
        sd = [None, None]
        for c in range(nch):
            b = c & 1
            nb = b ^ 1
            ld[b][0].wait()
            ld[b][1].wait()
            sd[b] = pltpu.async_copy(rows[b], xs_hbm.at[idx[b]], sems[b])
            if c >= 1 and c + 1 < nch:
                sd[nb].wait()
                sd[nb] = None
                ld[nb] = load(c + 1, nb)
        for d in sd:
            if d is not None:
                d.wait()

    return _sc_scatter


@functools.lru_cache(maxsize=None)
def _make_sc_gather():
    mesh = plsc.VectorSubcoreMesh(core_axis_name="c", subcore_axis_name="s")
    nch = _RPW // _GATH_CHUNK

    @functools.partial(
        pl.kernel, mesh=mesh,
        out_type=jax.ShapeDtypeStruct((N, D_MODEL), jnp.float32),
        scratch_types=[
            pltpu.VMEM((_RPW,), jnp.int32),
            pltpu.VMEM((_GATH_CHUNK, D_MODEL), jnp.float32),
            pltpu.VMEM((_GATH_CHUNK, D_MODEL), jnp.float32),
            pltpu.VMEM((_GATH_CHUNK, D_MODEL), jnp.float32),
            pltpu.SemaphoreType.DMA,
            pltpu.SemaphoreType.DMA,
            pltpu.SemaphoreType.DMA,
            pltpu.SemaphoreType.DMA,
            pltpu.SemaphoreType.DMA,
            pltpu.SemaphoreType.DMA,
        ],
    )
    def _sc_gather(enc_hbm, rank_hbm, out_hbm, idx_all, rows0, rows1, rows2,
                   sg0, sg1, sg2, so0, so1, so2):
        wid = lax.axis_index("s") * _NC + lax.axis_index("c")
        base = wid * _RPW
        rows = (rows0, rows1, rows2)
        semg = (sg0, sg1, sg2)
        semo = (so0, so1, so2)
        pltpu.sync_copy(rank_hbm.at[pl.ds(base, _RPW)], idx_all)

        def gath(c, b):
            return pltpu.async_copy(
                enc_hbm.at[idx_all.at[pl.ds(c * _GATH_CHUNK, _GATH_CHUNK)]],
                rows[b], semg[b])

        def wrb(c, b):
            return pltpu.async_copy(
                rows[b],
                out_hbm.at[pl.ds(base + c * _GATH_CHUNK, _GATH_CHUNK)],
                semo[b])

        gd = [gath(j, j) for j in range(3)]
        wd = [None, None, None]
        for c in range(nch):
            b = c % 3
            gd[b].wait()
            wd[b] = wrb(c, b)
            if c + 3 < nch:
                wd[b].wait()
                wd[b] = None
                gd[b] = gath(c + 3, b)
        for d in wd:
            if d is not None:
                d.wait()

    return _sc_gather


def _mlp_body(eid_ref, xs_ref, w0_ref, b0_ref, w1_ref, b1_ref, out_ref):
    del eid_ref
    xt = xs_ref[...].astype(jnp.bfloat16)
    h = jnp.maximum(
        lax.dot_general(xt, w0_ref[0], (((1,), (0,)), ((), ())),
                        preferred_element_type=jnp.float32) + b0_ref[0], 0.0)
    out_ref[...] = lax.dot_general(
        h.astype(jnp.bfloat16), w1_ref[0], (((1,), (0,)), ((), ())),
        preferred_element_type=jnp.float32) + b1_ref[0]


def _run_mlp(expert_ids, xs, w0s, b0s, w1s, b1s):
    grid_spec = pltpu.PrefetchScalarGridSpec(
        num_scalar_prefetch=1,
        grid=(NT,),
        in_specs=[
            pl.BlockSpec((TILE, D_IN), lambda i, eid: (i, 0)),
            pl.BlockSpec((1, D_IN, D_HID), lambda i, eid: (eid[i], 0, 0)),
            pl.BlockSpec((1, 1, D_HID), lambda i, eid: (eid[i], 0, 0)),
            pl.BlockSpec((1, D_HID, D_MODEL), lambda i, eid: (eid[i], 0, 0)),
            pl.BlockSpec((1, 1, D_MODEL), lambda i, eid: (eid[i], 0, 0)),
        ],
        out_specs=pl.BlockSpec((TILE, D_MODEL), lambda i, eid: (i, 0)),
    )
    return pl.pallas_call(
        _mlp_body,
        grid_spec=grid_spec,
        out_shape=jax.ShapeDtypeStruct((M, D_MODEL), jnp.float32),
        compiler_params=pltpu.CompilerParams(
            dimension_semantics=("parallel",)),
    )(expert_ids, xs, w0s, b0s, w1s, b1s)


def kernel(x, types, W0_0, b0_0, W1_0, b1_0, W0_1, b0_1, W1_1, b1_1):
    types2d = types.astype(jnp.int32).reshape(_R, _C)
    rank2d, c0p = _compute_rank(types2d)
    rank = rank2d.reshape(N)
    expert_ids = (jnp.arange(NT, dtype=jnp.int32) * TILE >= c0p[0, 0]
                  ).astype(jnp.int32)
    xs = _make_sc_scatter()(x, rank)
    w0s = jnp.stack([W0_0, W0_1]).astype(jnp.bfloat16)
    b0s = jnp.stack([b0_0, b0_1]).reshape(2, 1, D_HID)
    w1s = jnp.stack([W1_0, W1_1]).astype(jnp.bfloat16)
    b1s = jnp.stack([b1_0, b1_1]).reshape(2, 1, D_MODEL)
    enc = _run_mlp(expert_ids, xs, w0s, b0s, w1s, b1s)
    return _make_sc_gather()(enc, rank)


# speedup vs baseline: 1.0055x; 1.0027x over previous
import functools

import jax
import jax.numpy as jnp
from jax import lax
from jax.experimental import pallas as pl
from jax.experimental.pallas import tpu as pltpu
from jax.experimental.pallas import tpu_sc as plsc

N = 8192
D_IN = 512
D_HID = 2048
D_MODEL = 2048
TILE = 512
NT = N // TILE + 1
M = NT * TILE

_R = 64
_C = 128


def _rank_body(t_ref, rank_ref, c0p_ref):
    t = t_ref[...]
    is0 = (t == 0).astype(jnp.float32)
    k = lax.broadcasted_iota(jnp.int32, (_C, _C), 0)
    j = lax.broadcasted_iota(jnp.int32, (_C, _C), 1)
    upper = (k <= j).astype(jnp.float32)
    cum_lane = lax.dot_general(is0, upper, (((1,), (0,)), ((), ())),
                               preferred_element_type=jnp.float32)
    row_tot = cum_lane[:, _C - 1:_C]
    i2 = lax.broadcasted_iota(jnp.int32, (_R, _R), 0)
    j2 = lax.broadcasted_iota(jnp.int32, (_R, _R), 1)
    lower = (j2 < i2).astype(jnp.float32)
    row_off = lax.dot_general(lower, row_tot, (((1,), (0,)), ((), ())),
                              preferred_element_type=jnp.float32)
    cum0 = (cum_lane + row_off).astype(jnp.int32)
    c0 = jnp.sum(is0).astype(jnp.int32)
    c0p = ((c0 + TILE - 1) // TILE) * TILE
    gi = (lax.broadcasted_iota(jnp.int32, (_R, _C), 0) * _C
          + lax.broadcasted_iota(jnp.int32, (_R, _C), 1))
    rank_ref[...] = jnp.where(t == 0, cum0 - 1, c0p + gi - cum0)
    c0p_ref[0, 0] = c0p


def _compute_rank(types2d):
    return pl.pallas_call(
        _rank_body,
        out_shape=(jax.ShapeDtypeStruct((_R, _C), jnp.int32),
                   jax.ShapeDtypeStruct((1, 1), jnp.int32)),
        out_specs=(pl.BlockSpec(memory_space=pltpu.VMEM),
                   pl.BlockSpec(memory_space=pltpu.SMEM)),
    )(types2d)


_NC = 2
_NS = 16
_NW = _NC * _NS
_RPW = N // _NW
_SCAT_CHUNK = 64
_GATH_CHUNK = 16

@functools.lru_cache(maxsize=None)
def _make_sc_scatter():
    mesh = plsc.VectorSubcoreMesh(core_axis_name="c", subcore_axis_name="s")

    @functools.partial(
        pl.kernel, mesh=mesh,
        out_type=jax.ShapeDtypeStruct((M, D_IN), jnp.float32),
        scratch_types=[
            pltpu.VMEM((_SCAT_CHUNK,), jnp.int32),
            pltpu.VMEM((_SCAT_CHUNK,), jnp.int32),
            pltpu.VMEM((_SCAT_CHUNK, D_IN), jnp.float32),
            pltpu.VMEM((_SCAT_CHUNK, D_IN), jnp.float32),
            pltpu.SemaphoreType.DMA,
            pltpu.SemaphoreType.DMA,
            pltpu.SemaphoreType.DMA,
            pltpu.SemaphoreType.DMA,
            pltpu.SemaphoreType.DMA,
            pltpu.SemaphoreType.DMA,
        ],
    )
    def _sc_scatter(x_hbm, rank_hbm, xs_hbm, idx0, idx1, rows0, rows1,
                    si0, si1, sx0, sx1, ss0, ss1):
        wid = lax.axis_index("s") * _NC + lax.axis_index("c")
        base = wid * _RPW
        nch = _RPW // _SCAT_CHUNK
        idx = (idx0, idx1)
        rows = (rows0, rows1)
        semi = (si0, si1)
        semx = (sx0, sx1)
        sems = (ss0, ss1)

        def load(c, b):
            off = base + c * _SCAT_CHUNK
            return (
                pltpu.async_copy(rank_hbm.at[pl.ds(off, _SCAT_CHUNK)],
                                 idx[b], semi[b]),
                pltpu.async_copy(x_hbm.at[pl.ds(off, _SCAT_CHUNK)],
                                 rows[b], semx[b]),
            )

        ld = [load(0, 0), load(1, 1)]
        sd = [None, None]
        for c in range(nch):
            b = c & 1
            nb = b ^ 1
            ld[b][0].wait()
            ld[b][1].wait()
            sd[b] = pltpu.async_copy(rows[b], xs_hbm.at[idx[b]], sems[b])
            if c >= 1 and c + 1 < nch:
                sd[nb].wait()
                sd[nb] = None
                ld[nb] = load(c + 1, nb)
        for d in sd:
            if d is not None:
                d.wait()

    return _sc_scatter


@functools.lru_cache(maxsize=None)
def _make_sc_gather():
    mesh = plsc.VectorSubcoreMesh(core_axis_name="c", subcore_axis_name="s")
    nch = _RPW // _GATH_CHUNK

    @functools.partial(
        pl.kernel, mesh=mesh,
        out_type=jax.ShapeDtypeStruct((N, D_MODEL), jnp.float32),
        scratch_types=[
            pltpu.VMEM((_RPW,), jnp.int32),
            pltpu.VMEM((_GATH_CHUNK, D_MODEL), jnp.float32),
            pltpu.VMEM((_GATH_CHUNK, D_MODEL), jnp.float32),
            pltpu.VMEM((_GATH_CHUNK, D_MODEL), jnp.float32),
            pltpu.SemaphoreType.DMA,
            pltpu.SemaphoreType.DMA,
            pltpu.SemaphoreType.DMA,
            pltpu.SemaphoreType.DMA,
            pltpu.SemaphoreType.DMA,
            pltpu.SemaphoreType.DMA,
        ],
    )
    def _sc_gather(enc_hbm, rank_hbm, out_hbm, idx_all, rows0, rows1, rows2,
                   sg0, sg1, sg2, so0, so1, so2):
        wid = lax.axis_index("s") * _NC + lax.axis_index("c")
        base = wid * _RPW
        rows = (rows0, rows1, rows2)
        semg = (sg0, sg1, sg2)
        semo = (so0, so1, so2)
        pltpu.sync_copy(rank_hbm.at[pl.ds(base, _RPW)], idx_all)

        def gath(c, b):
            return pltpu.async_copy(
                enc_hbm.at[idx_all.at[pl.ds(c * _GATH_CHUNK, _GATH_CHUNK)]],
                rows[b], semg[b])

        def wrb(c, b):
            return pltpu.async_copy(
                rows[b],
                out_hbm.at[pl.ds(base + c * _GATH_CHUNK, _GATH_CHUNK)],
                semo[b])

        gd = [gath(j, j) for j in range(3)]
        wd = [None, None, None]
        for c in range(nch):
            b = c % 3
            gd[b].wait()
            wd[b] = wrb(c, b)
            if c + 3 < nch:
                wd[b].wait()
                wd[b] = None
                gd[b] = gath(c + 3, b)
        for d in wd:
            if d is not None:
                d.wait()

    return _sc_gather


def _mlp_body(eid_ref, xs_ref, w0_ref, b0_ref, w1_ref, b1_ref, out_ref):
    del eid_ref
    xt = xs_ref[...].astype(jnp.bfloat16)
    h = jnp.maximum(
        lax.dot_general(xt, w0_ref[0], (((1,), (0,)), ((), ())),
                        preferred_element_type=jnp.float32) + b0_ref[0], 0.0)
    out_ref[...] = lax.dot_general(
        h.astype(jnp.bfloat16), w1_ref[0], (((1,), (0,)), ((), ())),
        preferred_element_type=jnp.float32) + b1_ref[0]


def _run_mlp(expert_ids, xs, w0s, b0s, w1s, b1s):
    grid_spec = pltpu.PrefetchScalarGridSpec(
        num_scalar_prefetch=1,
        grid=(NT,),
        in_specs=[
            pl.BlockSpec((TILE, D_IN), lambda i, eid: (i, 0)),
            pl.BlockSpec((1, D_IN, D_HID), lambda i, eid: (eid[i], 0, 0)),
            pl.BlockSpec((1, 1, D_HID), lambda i, eid: (eid[i], 0, 0)),
            pl.BlockSpec((1, D_HID, D_MODEL), lambda i, eid: (eid[i], 0, 0)),
            pl.BlockSpec((1, 1, D_MODEL), lambda i, eid: (eid[i], 0, 0)),
        ],
        out_specs=pl.BlockSpec((TILE, D_MODEL), lambda i, eid: (i, 0)),
    )
    return pl.pallas_call(
        _mlp_body,
        grid_spec=grid_spec,
        out_shape=jax.ShapeDtypeStruct((M, D_MODEL), jnp.float32),
        compiler_params=pltpu.CompilerParams(
            dimension_semantics=("parallel",)),
    )(expert_ids, xs, w0s, b0s, w1s, b1s)


def kernel(x, types, W0_0, b0_0, W1_0, b1_0, W0_1, b0_1, W1_1, b1_1):
    types2d = types.astype(jnp.int32).reshape(_R, _C)
    rank2d, c0p = _compute_rank(types2d)
    rank = rank2d.reshape(N)
    expert_ids = (jnp.arange(NT, dtype=jnp.int32) * TILE >= c0p[0, 0]
                  ).astype(jnp.int32)
    w0s = jnp.stack([W0_0, W0_1]).astype(jnp.bfloat16)
    b0s = jnp.stack([b0_0, b0_1]).reshape(2, 1, D_HID)
    w1s = jnp.stack([W1_0, W1_1]).astype(jnp.bfloat16)
    b1s = jnp.stack([b1_0, b1_1]).reshape(2, 1, D_MODEL)
    xs = _make_sc_scatter()(x, rank)
    enc = _run_mlp(expert_ids, xs, w0s, b0s, w1s, b1s)
    return _make_sc_gather()(enc, rank)
